# Initial kernel scaffold; baseline (speedup 1.0000x reference)
import jax
import jax.numpy as jnp
from jax.experimental import pallas as pl


def kernel(x, edge_index, W1l, b1l, W1r, W2l, b2l, W2r):
    raise NotImplementedError("write your pallas kernel here")


# speedup vs baseline: 1.1442x; 1.1442x over previous
import functools

import jax
import jax.numpy as jnp
from jax.experimental import pallas as pl
from jax.experimental.pallas import tpu as pltpu

N_NODES = 10000
NF = 128


def _sage_dense_body(with_transition, x_ref, p_ref, c_ref, wl_ref, bl_ref,
                     wr_ref, o_ref):
    agg = jnp.sum(p_ref[...], axis=0)
    cnt = jnp.sum(c_ref[...], axis=0)[:, 0:1]
    agg = agg / jnp.maximum(cnt, 1.0)
    x = x_ref[...]
    dn = (((1,), (1,)), ((), ()))
    out = (jax.lax.dot_general(agg, wl_ref[...], dn,
                               preferred_element_type=jnp.float32,
                               precision=jax.lax.Precision.HIGHEST)
           + bl_ref[...]
           + jax.lax.dot_general(x, wr_ref[...], dn,
                                 preferred_element_type=jnp.float32,
                                 precision=jax.lax.Precision.HIGHEST))
    nrm = jnp.sqrt(jnp.sum(out * out, axis=1, keepdims=True))
    out = out / jnp.maximum(nrm, 1e-12)
    if with_transition:
        h = jnp.maximum(out, 0.0)
        mean = jnp.mean(h, axis=0, keepdims=True)
        var = jnp.mean(h * h, axis=0, keepdims=True) - mean * mean
        out = (h - mean) / jnp.sqrt(var + 1e-5)
    o_ref[...] = out


def _sage_dense(x, p, c, wl, bl, wr, with_transition):
    return pl.pallas_call(
        functools.partial(_sage_dense_body, with_transition),
        out_shape=jax.ShapeDtypeStruct((x.shape[0], wl.shape[0]), jnp.float32),
    )(x, p, c, wl, bl, wr)


def kernel(x, edge_index, W1l, b1l, W1r, W2l, b2l, W2r):
    src = edge_index[0]
    dst = edge_index[1]
    n = x.shape[0]
    p1 = jax.ops.segment_sum(jnp.take(x, src, axis=0), dst, num_segments=n)
    cnt = jax.ops.segment_sum(jnp.ones((src.shape[0], 16), jnp.float32), dst,
                              num_segments=n)
    b1 = jnp.reshape(b1l, (1, -1))
    b2 = jnp.reshape(b2l, (1, -1))
    h = _sage_dense(x, p1[None], cnt[None], W1l, b1, W1r, True)
    p2 = jax.ops.segment_sum(jnp.take(h, src, axis=0), dst, num_segments=n)
    return _sage_dense(h, p2[None], cnt[None], W2l, b2, W2r, False)


# speedup vs baseline: 7.8576x; 6.8671x over previous
import functools

import jax
import jax.numpy as jnp
from jax import lax
from jax.experimental import pallas as pl
from jax.experimental.pallas import tpu as pltpu
from jax.experimental.pallas import tpu_sc as plsc

N = 10000
NF = 128
E = 320000
NC = 2
NS = 16
NW = NC * NS
EW = E // NW
CH = 80
NCHUNK = EW // CH
GP = 25
NG = NCHUNK // GP
RB = 632
RB_LAST = N - (NS - 1) * RB
NBLK = 10
BLK = N // NBLK


def _per_tile_rows(s, fn):
    @pl.when(s < NS - 1)
    def _():
        fn(pl.multiple_of(s * RB, 8), RB)

    @pl.when(s == NS - 1)
    def _():
        fn((NS - 1) * RB, RB_LAST)


def _sc_segsum_body(x_hbm, src_hbm, dst_hbm, z_hbm, pagg,
                    sidx, didx, rows_a, rows_b, acc, sem_a, sem_b):
    c = lax.axis_index("c")
    s = lax.axis_index("s")
    wid = s * NC + c

    _per_tile_rows(s, lambda rb, nr: pltpu.sync_copy(
        z_hbm.at[pl.ds(0, nr)], acc.at[pl.ds(rb, nr)]))
    plsc.subcore_barrier()

    def wait_for(rows, sem):
        pltpu.make_async_copy(x_hbm.at[pl.ds(0, CH)], rows, sem).wait()

    def group(g, _):
        pltpu.sync_copy(src_hbm.at[wid, g], sidx)
        pltpu.sync_copy(dst_hbm.at[wid, g], didx)
        pltpu.async_copy(x_hbm.at[sidx.at[0]], rows_a, sem_a)

        def pair(k, _):
            j = k * 2
            wait_for(rows_a, sem_a)
            pltpu.async_copy(x_hbm.at[sidx.at[j + 1]], rows_b, sem_b)
            pltpu.sync_copy(rows_a, acc.at[didx.at[j]], add=True)
            wait_for(rows_b, sem_b)
            pltpu.async_copy(x_hbm.at[sidx.at[j + 2]], rows_a, sem_a)
            pltpu.sync_copy(rows_b, acc.at[didx.at[j + 1]], add=True)
            return 0

        lax.fori_loop(0, (GP - 1) // 2, pair, 0)
        wait_for(rows_a, sem_a)
        pltpu.sync_copy(rows_a, acc.at[didx.at[GP - 1]], add=True)
        return 0

    lax.fori_loop(0, NG, group, 0)
    plsc.subcore_barrier()

    _per_tile_rows(s, lambda rb, nr: pltpu.sync_copy(
        acc.at[pl.ds(rb, nr)], pagg.at[c, pl.ds(rb, nr)]))


def _sc_segsum(x, src4, dst4, z128):
    fn = pl.kernel(
        _sc_segsum_body,
        out_type=jax.ShapeDtypeStruct((NC, N, NF), jnp.float32),
        mesh=plsc.VectorSubcoreMesh(core_axis_name="c", subcore_axis_name="s"),
        scratch_types=[
            pltpu.VMEM((GP, CH), jnp.int32),
            pltpu.VMEM((GP, CH), jnp.int32),
            pltpu.VMEM((CH, NF), jnp.float32),
            pltpu.VMEM((CH, NF), jnp.float32),
            pltpu.VMEM_SHARED((N, NF), jnp.float32),
            pltpu.SemaphoreType.DMA,
            pltpu.SemaphoreType.DMA,
        ],
    )
    return fn(x, src4, dst4, z128)


def _sc_cnt_body(dst_hbm, ones_hbm, z_hbm, pcnt, didx, ones_v, cacc, _sem):
    c = lax.axis_index("c")
    s = lax.axis_index("s")
    wid = s * NC + c

    pltpu.sync_copy(ones_hbm, ones_v)
    _per_tile_rows(s, lambda rb, nr: pltpu.sync_copy(
        z_hbm.at[pl.ds(0, nr)], cacc.at[pl.ds(rb, nr)]))
    plsc.subcore_barrier()

    def group(g, _):
        pltpu.sync_copy(dst_hbm.at[wid, g], didx)

        def chunk(j, _):
            pltpu.sync_copy(ones_v, cacc.at[didx.at[j]], add=True)
            return 0

        return lax.fori_loop(0, GP, chunk, 0)

    lax.fori_loop(0, NG, group, 0)
    plsc.subcore_barrier()

    _per_tile_rows(s, lambda rb, nr: pltpu.sync_copy(
        cacc.at[pl.ds(rb, nr)], pcnt.at[c, pl.ds(rb, nr)]))


def _sc_cnt(dst4, ones128, z128):
    fn = pl.kernel(
        _sc_cnt_body,
        out_type=jax.ShapeDtypeStruct((NC, N, NF), jnp.float32),
        mesh=plsc.VectorSubcoreMesh(core_axis_name="c", subcore_axis_name="s"),
        scratch_types=[
            pltpu.VMEM((GP, CH), jnp.int32),
            pltpu.VMEM((CH, NF), jnp.float32),
            pltpu.VMEM_SHARED((N, NF), jnp.float32),
            pltpu.SemaphoreType.DMA,
        ],
    )
    return fn(dst4, ones128, z128)


def _mm_t(a, w):
    return jax.lax.dot_general(a, w, (((1,), (1,)), ((), ())),
                               preferred_element_type=jnp.float32,
                               precision=jax.lax.Precision.HIGHEST)


def _tc1_body(x_ref, p_ref, c_ref, wl_ref, bl_ref, wr_ref,
              h_ref, st_ref, sacc):
    i = pl.program_id(0)
    cnt = jnp.sum(c_ref[...], axis=0)[:, 0:1]
    agg = jnp.sum(p_ref[...], axis=0) / jnp.maximum(cnt, 1.0)
    out = _mm_t(agg, wl_ref[...]) + bl_ref[...] + _mm_t(x_ref[...], wr_ref[...])
    nrm = jnp.sqrt(jnp.sum(out * out, axis=1, keepdims=True))
    h = jnp.maximum(out / jnp.maximum(nrm, 1e-12), 0.0)
    h_ref[...] = h
    s0 = jnp.sum(h, axis=0, keepdims=True)
    s1 = jnp.sum(h * h, axis=0, keepdims=True)
    upd = jnp.concatenate([s0, s1, jnp.zeros((6, NF), jnp.float32)], axis=0)

    @pl.when(i == 0)
    def _():
        sacc[...] = upd

    @pl.when(i > 0)
    def _():
        sacc[...] = sacc[...] + upd

    @pl.when(i == NBLK - 1)
    def _():
        st_ref[...] = sacc[...]


def _tc_layer1(x, p, c, wl, bl, wr):
    return pl.pallas_call(
        _tc1_body,
        grid=(NBLK,),
        in_specs=[
            pl.BlockSpec((BLK, NF), lambda i: (i, 0)),
            pl.BlockSpec((NC, BLK, NF), lambda i: (0, i, 0)),
            pl.BlockSpec((NC, BLK, NF), lambda i: (0, i, 0)),
            pl.BlockSpec((NF, NF), lambda i: (0, 0)),
            pl.BlockSpec((1, NF), lambda i: (0, 0)),
            pl.BlockSpec((NF, NF), lambda i: (0, 0)),
        ],
        out_specs=[
            pl.BlockSpec((BLK, NF), lambda i: (i, 0)),
            pl.BlockSpec((8, NF), lambda i: (0, 0)),
        ],
        out_shape=[
            jax.ShapeDtypeStruct((N, NF), jnp.float32),
            jax.ShapeDtypeStruct((8, NF), jnp.float32),
        ],
        scratch_shapes=[pltpu.VMEM((8, NF), jnp.float32)],
    )(x, p, c, wl, bl, wr)


def _tc2_body(h_ref, p_ref, c_ref, st_ref, wl_ref, bl_ref, wr_ref, o_ref):
    stats = st_ref[...]
    mean = stats[0:1, :] * (1.0 / N)
    var = stats[1:2, :] * (1.0 / N) - mean * mean
    inv = 1.0 / jnp.sqrt(var + 1e-5)
    hbn = (h_ref[...] - mean) * inv
    cnt = jnp.sum(c_ref[...], axis=0)[:, 0:1]
    aggr = jnp.sum(p_ref[...], axis=0) / jnp.maximum(cnt, 1.0)
    aggbn = jnp.where(cnt > 0.0, (aggr - mean) * inv, 0.0)
    out = _mm_t(aggbn, wl_ref[...]) + bl_ref[...] + _mm_t(hbn, wr_ref[...])
    nrm = jnp.sqrt(jnp.sum(out * out, axis=1, keepdims=True))
    o_ref[...] = out / jnp.maximum(nrm, 1e-12)


def _tc_layer2(h, p, c, stats, wl, bl, wr):
    return pl.pallas_call(
        _tc2_body,
        grid=(NBLK,),
        in_specs=[
            pl.BlockSpec((BLK, NF), lambda i: (i, 0)),
            pl.BlockSpec((NC, BLK, NF), lambda i: (0, i, 0)),
            pl.BlockSpec((NC, BLK, NF), lambda i: (0, i, 0)),
            pl.BlockSpec((8, NF), lambda i: (0, 0)),
            pl.BlockSpec((NF, NF), lambda i: (0, 0)),
            pl.BlockSpec((1, NF), lambda i: (0, 0)),
            pl.BlockSpec((NF, NF), lambda i: (0, 0)),
        ],
        out_specs=pl.BlockSpec((BLK, NF), lambda i: (i, 0)),
        out_shape=jax.ShapeDtypeStruct((N, NF), jnp.float32),
    )(h, p, c, stats, wl, bl, wr)


def kernel(x, edge_index, W1l, b1l, W1r, W2l, b2l, W2r):
    src3 = edge_index[0].astype(jnp.int32).reshape(NW, NG, GP, CH)
    dst3 = edge_index[1].astype(jnp.int32).reshape(NW, NG, GP, CH)
    b1 = jnp.reshape(b1l, (1, -1))
    b2 = jnp.reshape(b2l, (1, -1))
    z128 = jnp.zeros((RB, NF), jnp.float32)
    ones128 = jnp.ones((CH, NF), jnp.float32)
    c1 = _sc_cnt(dst3, ones128, z128)
    p1 = _sc_segsum(x, src3, dst3, z128)
    h, stats = _tc_layer1(x, p1, c1, W1l, b1, W1r)
    p2 = _sc_segsum(h, src3, dst3, z128)
    return _tc_layer2(h, p2, c1, stats, W2l, b2, W2r)


# speedup vs baseline: 10.0210x; 1.2753x over previous
import functools

import jax
import jax.numpy as jnp
from jax import lax
from jax.experimental import pallas as pl
from jax.experimental.pallas import tpu as pltpu
from jax.experimental.pallas import tpu_sc as plsc

N = 10000
NF = 128
E = 320000
NC = 2
NS = 16
NW = NC * NS
EW = E // NW
CH = 80
NCHUNK = EW // CH
GP = 25
NG = NCHUNK // GP
RB = 632
RB_LAST = N - (NS - 1) * RB
NBLK = 10
BLK = N // NBLK


def _per_tile_rows(s, fn):
    @pl.when(s < NS - 1)
    def _():
        fn(pl.multiple_of(s * RB, 8), RB)

    @pl.when(s == NS - 1)
    def _():
        fn((NS - 1) * RB, RB_LAST)


def _sc_segsum_body(x_hbm, src_hbm, dst_hbm, z_hbm, pagg,
                    sidx, didx, rows_a, rows_b, rows_c, acc,
                    ga, gb, gc, sa, sb, sc):
    c = lax.axis_index("c")
    s = lax.axis_index("s")
    wid = s * NC + c

    _per_tile_rows(s, lambda rb, nr: pltpu.sync_copy(
        z_hbm.at[pl.ds(0, nr)], acc.at[pl.ds(rb, nr)]))
    plsc.subcore_barrier()

    def gwait(rows, sem):
        pltpu.make_async_copy(x_hbm.at[pl.ds(0, CH)], rows, sem).wait()

    def swait(rows, sem):
        pltpu.make_async_copy(rows, acc.at[didx.at[0]], sem).wait()

    def gather(j, rows, sem):
        pltpu.async_copy(x_hbm.at[sidx.at[j]], rows, sem)

    def scat(j, rows, sem):
        pltpu.async_copy(rows, acc.at[didx.at[j]], sem, add=True)

    def group(g, _):
        pltpu.sync_copy(src_hbm.at[wid, g], sidx)
        pltpu.sync_copy(dst_hbm.at[wid, g], didx)
        gather(0, rows_a, ga)
        gather(1, rows_b, gb)
        gwait(rows_a, ga)
        scat(0, rows_a, sa)
        gather(2, rows_c, gc)

        def tri(k, _):
            j = 3 * k + 1
            gwait(rows_b, gb)
            scat(j, rows_b, sb)
            swait(rows_a, sa)
            gather(j + 2, rows_a, ga)
            gwait(rows_c, gc)
            scat(j + 1, rows_c, sc)
            swait(rows_b, sb)
            gather(j + 3, rows_b, gb)
            gwait(rows_a, ga)
            scat(j + 2, rows_a, sa)
            swait(rows_c, sc)
            gather(j + 4, rows_c, gc)
            return 0

        lax.fori_loop(0, (GP - 4) // 3, tri, 0)
        gwait(rows_b, gb)
        scat(GP - 3, rows_b, sb)
        swait(rows_a, sa)
        gather(GP - 1, rows_a, ga)
        gwait(rows_c, gc)
        scat(GP - 2, rows_c, sc)
        swait(rows_b, sb)
        gwait(rows_a, ga)
        scat(GP - 1, rows_a, sa)
        swait(rows_c, sc)
        swait(rows_a, sa)
        return 0

    lax.fori_loop(0, NG, group, 0)
    plsc.subcore_barrier()

    _per_tile_rows(s, lambda rb, nr: pltpu.sync_copy(
        acc.at[pl.ds(rb, nr)], pagg.at[c, pl.ds(rb, nr)]))


def _sc_segsum(x, src4, dst4, z128):
    fn = pl.kernel(
        _sc_segsum_body,
        out_type=jax.ShapeDtypeStruct((NC, N, NF), jnp.float32),
        mesh=plsc.VectorSubcoreMesh(core_axis_name="c", subcore_axis_name="s"),
        scratch_types=[
            pltpu.VMEM((GP, CH), jnp.int32),
            pltpu.VMEM((GP, CH), jnp.int32),
            pltpu.VMEM((CH, NF), jnp.float32),
            pltpu.VMEM((CH, NF), jnp.float32),
            pltpu.VMEM((CH, NF), jnp.float32),
            pltpu.VMEM_SHARED((N, NF), jnp.float32),
            pltpu.SemaphoreType.DMA,
            pltpu.SemaphoreType.DMA,
            pltpu.SemaphoreType.DMA,
            pltpu.SemaphoreType.DMA,
            pltpu.SemaphoreType.DMA,
            pltpu.SemaphoreType.DMA,
        ],
    )
    return fn(x, src4, dst4, z128)


def _sc_cnt_body(dst_hbm, ones_hbm, z_hbm, pcnt, didx, ones_v, cacc, sem):
    c = lax.axis_index("c")
    s = lax.axis_index("s")
    wid = s * NC + c

    pltpu.sync_copy(ones_hbm, ones_v)
    _per_tile_rows(s, lambda rb, nr: pltpu.sync_copy(
        z_hbm.at[pl.ds(0, nr)], cacc.at[pl.ds(rb, nr)]))
    pltpu.sync_copy(dst_hbm.at[wid], didx)
    plsc.subcore_barrier()

    def fire(j, _):
        pltpu.async_copy(ones_v, cacc.at[didx.at[j]], sem, add=True)
        return 0

    def drain(j, _):
        pltpu.make_async_copy(ones_v, cacc.at[didx.at[0]], sem).wait()
        return 0

    def steady(j, _):
        drain(j, None)
        fire(j, None)
        return 0

    lax.fori_loop(0, 8, fire, 0)
    lax.fori_loop(8, NCHUNK, steady, 0)
    lax.fori_loop(0, 8, drain, 0)
    plsc.subcore_barrier()

    _per_tile_rows(s, lambda rb, nr: pltpu.sync_copy(
        cacc.at[pl.ds(rb, nr)], pcnt.at[c, pl.ds(rb, nr)]))


def _sc_cnt(dst4, ones128, z128):
    fn = pl.kernel(
        _sc_cnt_body,
        out_type=jax.ShapeDtypeStruct((NC, N, NF), jnp.float32),
        mesh=plsc.VectorSubcoreMesh(core_axis_name="c", subcore_axis_name="s"),
        scratch_types=[
            pltpu.VMEM((NCHUNK, CH), jnp.int32),
            pltpu.VMEM((CH, NF), jnp.float32),
            pltpu.VMEM_SHARED((N, NF), jnp.float32),
            pltpu.SemaphoreType.DMA,
        ],
    )
    return fn(dst4, ones128, z128)


def _mm_t(a, w):
    return jax.lax.dot_general(a, w, (((1,), (1,)), ((), ())),
                               preferred_element_type=jnp.float32,
                               precision=jax.lax.Precision.HIGHEST)


def _tc1_body(x_ref, p_ref, c_ref, wl_ref, bl_ref, wr_ref,
              h_ref, st_ref, sacc):
    i = pl.program_id(0)
    cnt = jnp.sum(c_ref[...], axis=0)[:, 0:1]
    agg = jnp.sum(p_ref[...], axis=0) / jnp.maximum(cnt, 1.0)
    out = _mm_t(agg, wl_ref[...]) + bl_ref[...] + _mm_t(x_ref[...], wr_ref[...])
    nrm = jnp.sqrt(jnp.sum(out * out, axis=1, keepdims=True))
    h = jnp.maximum(out / jnp.maximum(nrm, 1e-12), 0.0)
    h_ref[...] = h
    s0 = jnp.sum(h, axis=0, keepdims=True)
    s1 = jnp.sum(h * h, axis=0, keepdims=True)
    upd = jnp.concatenate([s0, s1, jnp.zeros((6, NF), jnp.float32)], axis=0)

    @pl.when(i == 0)
    def _():
        sacc[...] = upd

    @pl.when(i > 0)
    def _():
        sacc[...] = sacc[...] + upd

    @pl.when(i == NBLK - 1)
    def _():
        st_ref[...] = sacc[...]


def _tc_layer1(x, p, c, wl, bl, wr):
    return pl.pallas_call(
        _tc1_body,
        grid=(NBLK,),
        in_specs=[
            pl.BlockSpec((BLK, NF), lambda i: (i, 0)),
            pl.BlockSpec((NC, BLK, NF), lambda i: (0, i, 0)),
            pl.BlockSpec((NC, BLK, NF), lambda i: (0, i, 0)),
            pl.BlockSpec((NF, NF), lambda i: (0, 0)),
            pl.BlockSpec((1, NF), lambda i: (0, 0)),
            pl.BlockSpec((NF, NF), lambda i: (0, 0)),
        ],
        out_specs=[
            pl.BlockSpec((BLK, NF), lambda i: (i, 0)),
            pl.BlockSpec((8, NF), lambda i: (0, 0)),
        ],
        out_shape=[
            jax.ShapeDtypeStruct((N, NF), jnp.float32),
            jax.ShapeDtypeStruct((8, NF), jnp.float32),
        ],
        scratch_shapes=[pltpu.VMEM((8, NF), jnp.float32)],
    )(x, p, c, wl, bl, wr)


def _tc2_body(h_ref, p_ref, c_ref, st_ref, wl_ref, bl_ref, wr_ref, o_ref):
    stats = st_ref[...]
    mean = stats[0:1, :] * (1.0 / N)
    var = stats[1:2, :] * (1.0 / N) - mean * mean
    inv = 1.0 / jnp.sqrt(var + 1e-5)
    hbn = (h_ref[...] - mean) * inv
    cnt = jnp.sum(c_ref[...], axis=0)[:, 0:1]
    aggr = jnp.sum(p_ref[...], axis=0) / jnp.maximum(cnt, 1.0)
    aggbn = jnp.where(cnt > 0.0, (aggr - mean) * inv, 0.0)
    out = _mm_t(aggbn, wl_ref[...]) + bl_ref[...] + _mm_t(hbn, wr_ref[...])
    nrm = jnp.sqrt(jnp.sum(out * out, axis=1, keepdims=True))
    o_ref[...] = out / jnp.maximum(nrm, 1e-12)


def _tc_layer2(h, p, c, stats, wl, bl, wr):
    return pl.pallas_call(
        _tc2_body,
        grid=(NBLK,),
        in_specs=[
            pl.BlockSpec((BLK, NF), lambda i: (i, 0)),
            pl.BlockSpec((NC, BLK, NF), lambda i: (0, i, 0)),
            pl.BlockSpec((NC, BLK, NF), lambda i: (0, i, 0)),
            pl.BlockSpec((8, NF), lambda i: (0, 0)),
            pl.BlockSpec((NF, NF), lambda i: (0, 0)),
            pl.BlockSpec((1, NF), lambda i: (0, 0)),
            pl.BlockSpec((NF, NF), lambda i: (0, 0)),
        ],
        out_specs=pl.BlockSpec((BLK, NF), lambda i: (i, 0)),
        out_shape=jax.ShapeDtypeStruct((N, NF), jnp.float32),
    )(h, p, c, stats, wl, bl, wr)


def kernel(x, edge_index, W1l, b1l, W1r, W2l, b2l, W2r):
    src3 = edge_index[0].astype(jnp.int32).reshape(NW, NG, GP, CH)
    dst3 = edge_index[1].astype(jnp.int32).reshape(NW, NG, GP, CH)
    b1 = jnp.reshape(b1l, (1, -1))
    b2 = jnp.reshape(b2l, (1, -1))
    z128 = jnp.zeros((RB, NF), jnp.float32)
    ones128 = jnp.ones((CH, NF), jnp.float32)
    c1 = _sc_cnt(edge_index[1].astype(jnp.int32).reshape(NW, NCHUNK, CH),
                 ones128, z128)
    p1 = _sc_segsum(x, src3, dst3, z128)
    h, stats = _tc_layer1(x, p1, c1, W1l, b1, W1r)
    p2 = _sc_segsum(h, src3, dst3, z128)
    return _tc_layer2(h, p2, c1, stats, W2l, b2, W2r)


# speedup vs baseline: 10.5786x; 1.0556x over previous
import functools

import jax
import jax.numpy as jnp
from jax import lax
from jax.experimental import pallas as pl
from jax.experimental.pallas import tpu as pltpu
from jax.experimental.pallas import tpu_sc as plsc

N = 10000
NF = 128
E = 320000
NC = 2
NS = 16
NW = NC * NS
EW = E // NW
CH = 80
NCHUNK = EW // CH
GP = 25
NG = NCHUNK // GP
RB = 632
RB_LAST = N - (NS - 1) * RB
NBLK = 5
BLK = N // NBLK


def _per_tile_rows(s, fn):
    @pl.when(s < NS - 1)
    def _():
        fn(pl.multiple_of(s * RB, 8), RB)

    @pl.when(s == NS - 1)
    def _():
        fn((NS - 1) * RB, RB_LAST)


def _sc_segsum_body(x_hbm, src_hbm, dst_hbm, z_hbm, pagg,
                    sidx, didx, rows_a, rows_b, rows_c, acc,
                    ga, gb, gc, sa, sb, sc):
    c = lax.axis_index("c")
    s = lax.axis_index("s")
    wid = s * NC + c

    _per_tile_rows(s, lambda rb, nr: pltpu.sync_copy(
        z_hbm.at[pl.ds(0, nr)], acc.at[pl.ds(rb, nr)]))
    plsc.subcore_barrier()

    def gwait(rows, sem):
        pltpu.make_async_copy(x_hbm.at[pl.ds(0, CH)], rows, sem).wait()

    def swait(rows, sem):
        pltpu.make_async_copy(rows, acc.at[didx.at[0]], sem).wait()

    def gather(j, rows, sem):
        pltpu.async_copy(x_hbm.at[sidx.at[j]], rows, sem)

    def scat(j, rows, sem):
        pltpu.async_copy(rows, acc.at[didx.at[j]], sem, add=True)

    def group(g, _):
        pltpu.sync_copy(src_hbm.at[wid, g], sidx)
        pltpu.sync_copy(dst_hbm.at[wid, g], didx)
        gather(0, rows_a, ga)
        gather(1, rows_b, gb)
        gwait(rows_a, ga)
        scat(0, rows_a, sa)
        gather(2, rows_c, gc)

        def tri(k, _):
            j = 3 * k + 1
            gwait(rows_b, gb)
            scat(j, rows_b, sb)
            swait(rows_a, sa)
            gather(j + 2, rows_a, ga)
            gwait(rows_c, gc)
            scat(j + 1, rows_c, sc)
            swait(rows_b, sb)
            gather(j + 3, rows_b, gb)
            gwait(rows_a, ga)
            scat(j + 2, rows_a, sa)
            swait(rows_c, sc)
            gather(j + 4, rows_c, gc)
            return 0

        lax.fori_loop(0, (GP - 4) // 3, tri, 0)
        gwait(rows_b, gb)
        scat(GP - 3, rows_b, sb)
        swait(rows_a, sa)
        gather(GP - 1, rows_a, ga)
        gwait(rows_c, gc)
        scat(GP - 2, rows_c, sc)
        swait(rows_b, sb)
        gwait(rows_a, ga)
        scat(GP - 1, rows_a, sa)
        swait(rows_c, sc)
        swait(rows_a, sa)
        return 0

    lax.fori_loop(0, NG, group, 0)
    plsc.subcore_barrier()

    _per_tile_rows(s, lambda rb, nr: pltpu.sync_copy(
        acc.at[pl.ds(rb, nr)], pagg.at[c, pl.ds(rb, nr)]))


def _sc_segsum(x, src4, dst4, z128):
    fn = pl.kernel(
        _sc_segsum_body,
        out_type=jax.ShapeDtypeStruct((NC, N, NF), jnp.float32),
        mesh=plsc.VectorSubcoreMesh(core_axis_name="c", subcore_axis_name="s"),
        scratch_types=[
            pltpu.VMEM((GP, CH), jnp.int32),
            pltpu.VMEM((GP, CH), jnp.int32),
            pltpu.VMEM((CH, NF), jnp.float32),
            pltpu.VMEM((CH, NF), jnp.float32),
            pltpu.VMEM((CH, NF), jnp.float32),
            pltpu.VMEM_SHARED((N, NF), jnp.float32),
            pltpu.SemaphoreType.DMA,
            pltpu.SemaphoreType.DMA,
            pltpu.SemaphoreType.DMA,
            pltpu.SemaphoreType.DMA,
            pltpu.SemaphoreType.DMA,
            pltpu.SemaphoreType.DMA,
        ],
    )
    return fn(x, src4, dst4, z128)


def _sc_cnt_body(dst_hbm, ones_hbm, z_hbm, pcnt, didx, ones_v, cacc, sem):
    c = lax.axis_index("c")
    s = lax.axis_index("s")
    wid = s * NC + c

    pltpu.sync_copy(ones_hbm, ones_v)
    _per_tile_rows(s, lambda rb, nr: pltpu.sync_copy(
        z_hbm.at[pl.ds(0, nr)], cacc.at[pl.ds(rb, nr)]))
    pltpu.sync_copy(dst_hbm.at[wid], didx)
    plsc.subcore_barrier()

    def fire(j, _):
        pltpu.async_copy(ones_v, cacc.at[didx.at[j]], sem, add=True)
        return 0

    def drain(j, _):
        pltpu.make_async_copy(ones_v, cacc.at[didx.at[0]], sem).wait()
        return 0

    def steady(j, _):
        drain(j, None)
        fire(j, None)
        return 0

    lax.fori_loop(0, 8, fire, 0)
    lax.fori_loop(8, NCHUNK, steady, 0)
    lax.fori_loop(0, 8, drain, 0)
    plsc.subcore_barrier()

    _per_tile_rows(s, lambda rb, nr: pltpu.sync_copy(
        cacc.at[pl.ds(rb, nr)], pcnt.at[c, pl.ds(rb, nr)]))


def _sc_cnt(dst4, ones128, z128):
    fn = pl.kernel(
        _sc_cnt_body,
        out_type=jax.ShapeDtypeStruct((NC, N, NF), jnp.float32),
        mesh=plsc.VectorSubcoreMesh(core_axis_name="c", subcore_axis_name="s"),
        scratch_types=[
            pltpu.VMEM((NCHUNK, CH), jnp.int32),
            pltpu.VMEM((CH, NF), jnp.float32),
            pltpu.VMEM_SHARED((N, NF), jnp.float32),
            pltpu.SemaphoreType.DMA,
        ],
    )
    return fn(dst4, ones128, z128)


def _mm_t(a, w):
    return jax.lax.dot_general(a, w, (((1,), (1,)), ((), ())),
                               preferred_element_type=jnp.float32)


def _tc1_body(x_ref, p_ref, c_ref, wl_ref, bl_ref, wr_ref,
              h_ref, st_ref, sacc):
    i = pl.program_id(0)
    cnt = jnp.sum(c_ref[...], axis=0)[:, 0:1]
    agg = jnp.sum(p_ref[...], axis=0) / jnp.maximum(cnt, 1.0)
    out = _mm_t(agg, wl_ref[...]) + bl_ref[...] + _mm_t(x_ref[...], wr_ref[...])
    nrm = jnp.sqrt(jnp.sum(out * out, axis=1, keepdims=True))
    h = jnp.maximum(out / jnp.maximum(nrm, 1e-12), 0.0)
    h_ref[...] = h
    s0 = jnp.sum(h, axis=0, keepdims=True)
    s1 = jnp.sum(h * h, axis=0, keepdims=True)
    upd = jnp.concatenate([s0, s1, jnp.zeros((6, NF), jnp.float32)], axis=0)

    @pl.when(i == 0)
    def _():
        sacc[...] = upd

    @pl.when(i > 0)
    def _():
        sacc[...] = sacc[...] + upd

    @pl.when(i == NBLK - 1)
    def _():
        st_ref[...] = sacc[...]


def _tc_layer1(x, p, c, wl, bl, wr):
    return pl.pallas_call(
        _tc1_body,
        grid=(NBLK,),
        in_specs=[
            pl.BlockSpec((BLK, NF), lambda i: (i, 0)),
            pl.BlockSpec((NC, BLK, NF), lambda i: (0, i, 0)),
            pl.BlockSpec((NC, BLK, NF), lambda i: (0, i, 0)),
            pl.BlockSpec((NF, NF), lambda i: (0, 0)),
            pl.BlockSpec((1, NF), lambda i: (0, 0)),
            pl.BlockSpec((NF, NF), lambda i: (0, 0)),
        ],
        out_specs=[
            pl.BlockSpec((BLK, NF), lambda i: (i, 0)),
            pl.BlockSpec((8, NF), lambda i: (0, 0)),
        ],
        out_shape=[
            jax.ShapeDtypeStruct((N, NF), jnp.float32),
            jax.ShapeDtypeStruct((8, NF), jnp.float32),
        ],
        scratch_shapes=[pltpu.VMEM((8, NF), jnp.float32)],
    )(x, p, c, wl, bl, wr)


def _tc2_body(h_ref, p_ref, c_ref, st_ref, wl_ref, bl_ref, wr_ref, o_ref):
    stats = st_ref[...]
    mean = stats[0:1, :] * (1.0 / N)
    var = stats[1:2, :] * (1.0 / N) - mean * mean
    inv = 1.0 / jnp.sqrt(var + 1e-5)
    hbn = (h_ref[...] - mean) * inv
    cnt = jnp.sum(c_ref[...], axis=0)[:, 0:1]
    aggr = jnp.sum(p_ref[...], axis=0) / jnp.maximum(cnt, 1.0)
    aggbn = jnp.where(cnt > 0.0, (aggr - mean) * inv, 0.0)
    out = _mm_t(aggbn, wl_ref[...]) + bl_ref[...] + _mm_t(hbn, wr_ref[...])
    nrm = jnp.sqrt(jnp.sum(out * out, axis=1, keepdims=True))
    o_ref[...] = out / jnp.maximum(nrm, 1e-12)


def _tc_layer2(h, p, c, stats, wl, bl, wr):
    return pl.pallas_call(
        _tc2_body,
        grid=(NBLK,),
        in_specs=[
            pl.BlockSpec((BLK, NF), lambda i: (i, 0)),
            pl.BlockSpec((NC, BLK, NF), lambda i: (0, i, 0)),
            pl.BlockSpec((NC, BLK, NF), lambda i: (0, i, 0)),
            pl.BlockSpec((8, NF), lambda i: (0, 0)),
            pl.BlockSpec((NF, NF), lambda i: (0, 0)),
            pl.BlockSpec((1, NF), lambda i: (0, 0)),
            pl.BlockSpec((NF, NF), lambda i: (0, 0)),
        ],
        out_specs=pl.BlockSpec((BLK, NF), lambda i: (i, 0)),
        out_shape=jax.ShapeDtypeStruct((N, NF), jnp.float32),
    )(h, p, c, stats, wl, bl, wr)


def kernel(x, edge_index, W1l, b1l, W1r, W2l, b2l, W2r):
    src3 = edge_index[0].astype(jnp.int32).reshape(NW, NG, GP, CH)
    dst3 = edge_index[1].astype(jnp.int32).reshape(NW, NG, GP, CH)
    b1 = jnp.reshape(b1l, (1, -1))
    b2 = jnp.reshape(b2l, (1, -1))
    z128 = jnp.zeros((RB, NF), jnp.float32)
    ones128 = jnp.ones((CH, NF), jnp.float32)
    c1 = _sc_cnt(edge_index[1].astype(jnp.int32).reshape(NW, NCHUNK, CH),
                 ones128, z128)
    p1 = _sc_segsum(x, src3, dst3, z128)
    h, stats = _tc_layer1(x, p1, c1, W1l, b1, W1r)
    p2 = _sc_segsum(h, src3, dst3, z128)
    return _tc_layer2(h, p2, c1, stats, W2l, b2, W2r)


# speedup vs baseline: 10.6895x; 1.0105x over previous
import functools

import jax
import jax.numpy as jnp
from jax import lax
from jax.experimental import pallas as pl
from jax.experimental.pallas import tpu as pltpu
from jax.experimental.pallas import tpu_sc as plsc

N = 10000
NF = 128
E = 320000
NC = 2
NS = 16
NW = NC * NS
EW = E // NW
CH = 80
NCHUNK = EW // CH
GP = 25
NG = NCHUNK // GP
RB = 632
RB_LAST = N - (NS - 1) * RB
NBLK = 5
BLK = N // NBLK


def _per_tile_rows(s, fn):
    @pl.when(s < NS - 1)
    def _():
        fn(pl.multiple_of(s * RB, 8), RB)

    @pl.when(s == NS - 1)
    def _():
        fn((NS - 1) * RB, RB_LAST)


def _sc_segsum_body(with_cnt, *refs):
    if with_cnt:
        (x_hbm, src_hbm, dst_hbm, z_hbm, ones_hbm, pagg, pcnt,
         sidx, didx, rows_a, rows_b, rows_c, acc, ga, gb, gc, sa, sb, sc) = refs
    else:
        (x_hbm, src_hbm, dst_hbm, z_hbm, pagg,
         sidx, didx, rows_a, rows_b, rows_c, acc, ga, gb, gc, sa, sb, sc) = refs
    c = lax.axis_index("c")
    s = lax.axis_index("s")
    wid = s * NC + c

    _per_tile_rows(s, lambda rb, nr: pltpu.sync_copy(
        z_hbm.at[pl.ds(0, nr)], acc.at[pl.ds(rb, nr)]))
    plsc.subcore_barrier()

    if with_cnt:
        pltpu.sync_copy(ones_hbm, rows_c)

        def cgroup(g, _):
            pltpu.sync_copy(dst_hbm.at[wid, g], didx)

            def fire(j, _):
                pltpu.async_copy(rows_c, acc.at[didx.at[j]], sa, add=True)
                return 0

            def steady(j, _):
                pltpu.make_async_copy(rows_c, acc.at[didx.at[0]], sa).wait()
                pltpu.async_copy(rows_c, acc.at[didx.at[j]], sa, add=True)
                return 0

            def cdrain(j, _):
                pltpu.make_async_copy(rows_c, acc.at[didx.at[0]], sa).wait()
                return 0

            lax.fori_loop(0, 8, fire, 0)
            lax.fori_loop(8, GP, steady, 0)
            lax.fori_loop(0, 8, cdrain, 0)
            return 0

        lax.fori_loop(0, NG, cgroup, 0)
        plsc.subcore_barrier()
        _per_tile_rows(s, lambda rb, nr: pltpu.sync_copy(
            acc.at[pl.ds(rb, nr)], pcnt.at[c, pl.ds(rb, nr)]))
        plsc.subcore_barrier()
        _per_tile_rows(s, lambda rb, nr: pltpu.sync_copy(
            z_hbm.at[pl.ds(0, nr)], acc.at[pl.ds(rb, nr)]))
        plsc.subcore_barrier()

    def gwait(rows, sem):
        pltpu.make_async_copy(x_hbm.at[pl.ds(0, CH)], rows, sem).wait()

    def swait(rows, sem):
        pltpu.make_async_copy(rows, acc.at[didx.at[0]], sem).wait()

    def gather(j, rows, sem):
        pltpu.async_copy(x_hbm.at[sidx.at[j]], rows, sem)

    def scat(j, rows, sem):
        pltpu.async_copy(rows, acc.at[didx.at[j]], sem, add=True)

    def group(g, _):
        pltpu.sync_copy(src_hbm.at[wid, g], sidx)
        pltpu.sync_copy(dst_hbm.at[wid, g], didx)
        gather(0, rows_a, ga)
        gather(1, rows_b, gb)
        gwait(rows_a, ga)
        scat(0, rows_a, sa)
        gather(2, rows_c, gc)

        def tri(k, _):
            j = 3 * k + 1
            gwait(rows_b, gb)
            scat(j, rows_b, sb)
            swait(rows_a, sa)
            gather(j + 2, rows_a, ga)
            gwait(rows_c, gc)
            scat(j + 1, rows_c, sc)
            swait(rows_b, sb)
            gather(j + 3, rows_b, gb)
            gwait(rows_a, ga)
            scat(j + 2, rows_a, sa)
            swait(rows_c, sc)
            gather(j + 4, rows_c, gc)
            return 0

        lax.fori_loop(0, (GP - 4) // 3, tri, 0)
        gwait(rows_b, gb)
        scat(GP - 3, rows_b, sb)
        swait(rows_a, sa)
        gather(GP - 1, rows_a, ga)
        gwait(rows_c, gc)
        scat(GP - 2, rows_c, sc)
        swait(rows_b, sb)
        gwait(rows_a, ga)
        scat(GP - 1, rows_a, sa)
        swait(rows_c, sc)
        swait(rows_a, sa)
        return 0

    lax.fori_loop(0, NG, group, 0)
    plsc.subcore_barrier()

    _per_tile_rows(s, lambda rb, nr: pltpu.sync_copy(
        acc.at[pl.ds(rb, nr)], pagg.at[c, pl.ds(rb, nr)]))


def _sc_segsum(x, src4, dst4, z128, ones128=None):
    with_cnt = ones128 is not None
    if with_cnt:
        out_type = [jax.ShapeDtypeStruct((NC, N, NF), jnp.float32),
                    jax.ShapeDtypeStruct((NC, N, NF), jnp.float32)]
    else:
        out_type = jax.ShapeDtypeStruct((NC, N, NF), jnp.float32)
    fn = pl.kernel(
        functools.partial(_sc_segsum_body, with_cnt),
        out_type=out_type,
        mesh=plsc.VectorSubcoreMesh(core_axis_name="c", subcore_axis_name="s"),
        scratch_types=[
            pltpu.VMEM((GP, CH), jnp.int32),
            pltpu.VMEM((GP, CH), jnp.int32),
            pltpu.VMEM((CH, NF), jnp.float32),
            pltpu.VMEM((CH, NF), jnp.float32),
            pltpu.VMEM((CH, NF), jnp.float32),
            pltpu.VMEM_SHARED((N, NF), jnp.float32),
            pltpu.SemaphoreType.DMA,
            pltpu.SemaphoreType.DMA,
            pltpu.SemaphoreType.DMA,
            pltpu.SemaphoreType.DMA,
            pltpu.SemaphoreType.DMA,
            pltpu.SemaphoreType.DMA,
        ],
    )
    if with_cnt:
        return fn(x, src4, dst4, z128, ones128)
    return fn(x, src4, dst4, z128)


def _mm_t(a, w):
    return jax.lax.dot_general(a, w, (((1,), (1,)), ((), ())),
                               preferred_element_type=jnp.float32)


def _tc1_body(x_ref, p_ref, c_ref, wl_ref, bl_ref, wr_ref,
              h_ref, st_ref, sacc):
    i = pl.program_id(0)
    cnt = jnp.sum(c_ref[...], axis=0)[:, 0:1]
    agg = jnp.sum(p_ref[...], axis=0) / jnp.maximum(cnt, 1.0)
    out = _mm_t(agg, wl_ref[...]) + bl_ref[...] + _mm_t(x_ref[...], wr_ref[...])
    nrm = jnp.sqrt(jnp.sum(out * out, axis=1, keepdims=True))
    h = jnp.maximum(out / jnp.maximum(nrm, 1e-12), 0.0)
    h_ref[...] = h
    s0 = jnp.sum(h, axis=0, keepdims=True)
    s1 = jnp.sum(h * h, axis=0, keepdims=True)
    upd = jnp.concatenate([s0, s1, jnp.zeros((6, NF), jnp.float32)], axis=0)

    @pl.when(i == 0)
    def _():
        sacc[...] = upd

    @pl.when(i > 0)
    def _():
        sacc[...] = sacc[...] + upd

    @pl.when(i == NBLK - 1)
    def _():
        st_ref[...] = sacc[...]


def _tc_layer1(x, p, c, wl, bl, wr):
    return pl.pallas_call(
        _tc1_body,
        grid=(NBLK,),
        in_specs=[
            pl.BlockSpec((BLK, NF), lambda i: (i, 0)),
            pl.BlockSpec((NC, BLK, NF), lambda i: (0, i, 0)),
            pl.BlockSpec((NC, BLK, NF), lambda i: (0, i, 0)),
            pl.BlockSpec((NF, NF), lambda i: (0, 0)),
            pl.BlockSpec((1, NF), lambda i: (0, 0)),
            pl.BlockSpec((NF, NF), lambda i: (0, 0)),
        ],
        out_specs=[
            pl.BlockSpec((BLK, NF), lambda i: (i, 0)),
            pl.BlockSpec((8, NF), lambda i: (0, 0)),
        ],
        out_shape=[
            jax.ShapeDtypeStruct((N, NF), jnp.float32),
            jax.ShapeDtypeStruct((8, NF), jnp.float32),
        ],
        scratch_shapes=[pltpu.VMEM((8, NF), jnp.float32)],
    )(x, p, c, wl, bl, wr)


def _tc2_body(h_ref, p_ref, c_ref, st_ref, wl_ref, bl_ref, wr_ref, o_ref):
    stats = st_ref[...]
    mean = stats[0:1, :] * (1.0 / N)
    var = stats[1:2, :] * (1.0 / N) - mean * mean
    inv = 1.0 / jnp.sqrt(var + 1e-5)
    hbn = (h_ref[...] - mean) * inv
    cnt = jnp.sum(c_ref[...], axis=0)[:, 0:1]
    aggr = jnp.sum(p_ref[...], axis=0) / jnp.maximum(cnt, 1.0)
    aggbn = jnp.where(cnt > 0.0, (aggr - mean) * inv, 0.0)
    out = _mm_t(aggbn, wl_ref[...]) + bl_ref[...] + _mm_t(hbn, wr_ref[...])
    nrm = jnp.sqrt(jnp.sum(out * out, axis=1, keepdims=True))
    o_ref[...] = out / jnp.maximum(nrm, 1e-12)


def _tc_layer2(h, p, c, stats, wl, bl, wr):
    return pl.pallas_call(
        _tc2_body,
        grid=(NBLK,),
        in_specs=[
            pl.BlockSpec((BLK, NF), lambda i: (i, 0)),
            pl.BlockSpec((NC, BLK, NF), lambda i: (0, i, 0)),
            pl.BlockSpec((NC, BLK, NF), lambda i: (0, i, 0)),
            pl.BlockSpec((8, NF), lambda i: (0, 0)),
            pl.BlockSpec((NF, NF), lambda i: (0, 0)),
            pl.BlockSpec((1, NF), lambda i: (0, 0)),
            pl.BlockSpec((NF, NF), lambda i: (0, 0)),
        ],
        out_specs=pl.BlockSpec((BLK, NF), lambda i: (i, 0)),
        out_shape=jax.ShapeDtypeStruct((N, NF), jnp.float32),
    )(h, p, c, stats, wl, bl, wr)


def kernel(x, edge_index, W1l, b1l, W1r, W2l, b2l, W2r):
    src3 = edge_index[0].astype(jnp.int32).reshape(NW, NG, GP, CH)
    dst3 = edge_index[1].astype(jnp.int32).reshape(NW, NG, GP, CH)
    b1 = jnp.reshape(b1l, (1, -1))
    b2 = jnp.reshape(b2l, (1, -1))
    z128 = jnp.zeros((RB, NF), jnp.float32)
    ones128 = jnp.ones((CH, NF), jnp.float32)
    p1, c1 = _sc_segsum(x, src3, dst3, z128, ones128)
    h, stats = _tc_layer1(x, p1, c1, W1l, b1, W1r)
    p2 = _sc_segsum(h, src3, dst3, z128)
    return _tc_layer2(h, p2, c1, stats, W2l, b2, W2r)


# speedup vs baseline: 10.9257x; 1.0221x over previous
import functools

import jax
import jax.numpy as jnp
from jax import lax
from jax.experimental import pallas as pl
from jax.experimental.pallas import tpu as pltpu
from jax.experimental.pallas import tpu_sc as plsc

N = 10000
NF = 128
E = 320000
NC = 2
NS = 16
NW = NC * NS
EW = E // NW
CH = 80
NCHUNK = EW // CH
GP = 25
NG = NCHUNK // GP
RB = 632
RB_LAST = N - (NS - 1) * RB
NBLK = 5
BLK = N // NBLK


def _per_tile_rows(s, fn):
    @pl.when(s < NS - 1)
    def _():
        fn(pl.multiple_of(s * RB, 8), RB)

    @pl.when(s == NS - 1)
    def _():
        fn((NS - 1) * RB, RB_LAST)


def _sc_segsum_body(with_cnt, *refs):
    if with_cnt:
        (x_hbm, src_hbm, dst_hbm, z_hbm, ones_hbm, pagg, pcnt,
         sidx, didx, rows_a, rows_b, rows_c, acc, ga, gb, gc, sa, sb, sc) = refs
    else:
        (x_hbm, src_hbm, dst_hbm, z_hbm, pagg,
         sidx, didx, rows_a, rows_b, rows_c, acc, ga, gb, gc, sa, sb, sc) = refs
    c = lax.axis_index("c")
    s = lax.axis_index("s")
    wid = s * NC + c

    _per_tile_rows(s, lambda rb, nr: pltpu.sync_copy(
        z_hbm.at[pl.ds(0, nr)], acc.at[pl.ds(rb, nr)]))
    plsc.subcore_barrier()

    if with_cnt:
        pltpu.sync_copy(ones_hbm, rows_c)

        def cgroup(g, _):
            pltpu.sync_copy(dst_hbm.at[wid, g], didx)

            def fire(j, _):
                pltpu.async_copy(rows_c, acc.at[didx.at[j]], sa, add=True)
                return 0

            def steady(j, _):
                pltpu.make_async_copy(rows_c, acc.at[didx.at[0]], sa).wait()
                pltpu.async_copy(rows_c, acc.at[didx.at[j]], sa, add=True)
                return 0

            def cdrain(j, _):
                pltpu.make_async_copy(rows_c, acc.at[didx.at[0]], sa).wait()
                return 0

            lax.fori_loop(0, 8, fire, 0)
            lax.fori_loop(8, GP, steady, 0)
            lax.fori_loop(0, 8, cdrain, 0)
            return 0

        lax.fori_loop(0, NG, cgroup, 0)
        plsc.subcore_barrier()
        _per_tile_rows(s, lambda rb, nr: pltpu.sync_copy(
            acc.at[pl.ds(rb, nr)], pcnt.at[c, pl.ds(rb, nr)]))
        plsc.subcore_barrier()

    def gwait(rows, sem):
        pltpu.make_async_copy(x_hbm.at[pl.ds(0, CH)], rows, sem).wait()

    def swait(rows, sem):
        pltpu.make_async_copy(rows, acc.at[didx.at[0]], sem).wait()

    def gather(j, rows, sem):
        pltpu.async_copy(x_hbm.at[sidx.at[j]], rows, sem)

    def scat(j, rows, sem):
        pltpu.async_copy(rows, acc.at[didx.at[j]], sem, add=True)

    def group(g, _):
        pltpu.sync_copy(src_hbm.at[wid, g], sidx)
        pltpu.sync_copy(dst_hbm.at[wid, g], didx)
        gather(0, rows_a, ga)
        gather(1, rows_b, gb)
        gwait(rows_a, ga)
        scat(0, rows_a, sa)
        gather(2, rows_c, gc)

        def tri(k, _):
            j = 3 * k + 1
            gwait(rows_b, gb)
            scat(j, rows_b, sb)
            swait(rows_a, sa)
            gather(j + 2, rows_a, ga)
            gwait(rows_c, gc)
            scat(j + 1, rows_c, sc)
            swait(rows_b, sb)
            gather(j + 3, rows_b, gb)
            gwait(rows_a, ga)
            scat(j + 2, rows_a, sa)
            swait(rows_c, sc)
            gather(j + 4, rows_c, gc)
            return 0

        lax.fori_loop(0, (GP - 4) // 3, tri, 0)
        gwait(rows_b, gb)
        scat(GP - 3, rows_b, sb)
        swait(rows_a, sa)
        gather(GP - 1, rows_a, ga)
        gwait(rows_c, gc)
        scat(GP - 2, rows_c, sc)
        swait(rows_b, sb)
        gwait(rows_a, ga)
        scat(GP - 1, rows_a, sa)
        swait(rows_c, sc)
        swait(rows_a, sa)
        return 0

    lax.fori_loop(0, NG, group, 0)
    plsc.subcore_barrier()

    _per_tile_rows(s, lambda rb, nr: pltpu.sync_copy(
        acc.at[pl.ds(rb, nr)], pagg.at[c, pl.ds(rb, nr)]))


def _sc_segsum(x, src4, dst4, z128, ones128=None):
    with_cnt = ones128 is not None
    if with_cnt:
        out_type = [jax.ShapeDtypeStruct((NC, N, NF), jnp.float32),
                    jax.ShapeDtypeStruct((NC, N, NF), jnp.float32)]
    else:
        out_type = jax.ShapeDtypeStruct((NC, N, NF), jnp.float32)
    fn = pl.kernel(
        functools.partial(_sc_segsum_body, with_cnt),
        out_type=out_type,
        mesh=plsc.VectorSubcoreMesh(core_axis_name="c", subcore_axis_name="s"),
        scratch_types=[
            pltpu.VMEM((GP, CH), jnp.int32),
            pltpu.VMEM((GP, CH), jnp.int32),
            pltpu.VMEM((CH, NF), jnp.float32),
            pltpu.VMEM((CH, NF), jnp.float32),
            pltpu.VMEM((CH, NF), jnp.float32),
            pltpu.VMEM_SHARED((N, NF), jnp.float32),
            pltpu.SemaphoreType.DMA,
            pltpu.SemaphoreType.DMA,
            pltpu.SemaphoreType.DMA,
            pltpu.SemaphoreType.DMA,
            pltpu.SemaphoreType.DMA,
            pltpu.SemaphoreType.DMA,
        ],
    )
    if with_cnt:
        return fn(x, src4, dst4, z128, ones128)
    return fn(x, src4, dst4, z128)


def _mm_t(a, w):
    return jax.lax.dot_general(a, w, (((1,), (1,)), ((), ())),
                               preferred_element_type=jnp.float32)


def _tc1_body(x_ref, p_ref, c_ref, wl_ref, bl_ref, wr_ref,
              h_ref, st_ref, sacc):
    i = pl.program_id(0)
    cnt = jnp.sum(c_ref[...], axis=0)[:, 0:1]
    agg = (jnp.sum(p_ref[...], axis=0) - cnt) / jnp.maximum(cnt, 1.0)
    out = _mm_t(agg, wl_ref[...]) + bl_ref[...] + _mm_t(x_ref[...], wr_ref[...])
    nrm = jnp.sqrt(jnp.sum(out * out, axis=1, keepdims=True))
    h = jnp.maximum(out / jnp.maximum(nrm, 1e-12), 0.0)
    h_ref[...] = h
    s0 = jnp.sum(h, axis=0, keepdims=True)
    s1 = jnp.sum(h * h, axis=0, keepdims=True)
    upd = jnp.concatenate([s0, s1, jnp.zeros((6, NF), jnp.float32)], axis=0)

    @pl.when(i == 0)
    def _():
        sacc[...] = upd

    @pl.when(i > 0)
    def _():
        sacc[...] = sacc[...] + upd

    @pl.when(i == NBLK - 1)
    def _():
        st_ref[...] = sacc[...]


def _tc_layer1(x, p, c, wl, bl, wr):
    return pl.pallas_call(
        _tc1_body,
        grid=(NBLK,),
        in_specs=[
            pl.BlockSpec((BLK, NF), lambda i: (i, 0)),
            pl.BlockSpec((NC, BLK, NF), lambda i: (0, i, 0)),
            pl.BlockSpec((NC, BLK, NF), lambda i: (0, i, 0)),
            pl.BlockSpec((NF, NF), lambda i: (0, 0)),
            pl.BlockSpec((1, NF), lambda i: (0, 0)),
            pl.BlockSpec((NF, NF), lambda i: (0, 0)),
        ],
        out_specs=[
            pl.BlockSpec((BLK, NF), lambda i: (i, 0)),
            pl.BlockSpec((8, NF), lambda i: (0, 0)),
        ],
        out_shape=[
            jax.ShapeDtypeStruct((N, NF), jnp.float32),
            jax.ShapeDtypeStruct((8, NF), jnp.float32),
        ],
        scratch_shapes=[pltpu.VMEM((8, NF), jnp.float32)],
    )(x, p, c, wl, bl, wr)


def _tc2_body(h_ref, p_ref, c_ref, st_ref, wl_ref, bl_ref, wr_ref, o_ref):
    stats = st_ref[...]
    mean = stats[0:1, :] * (1.0 / N)
    var = stats[1:2, :] * (1.0 / N) - mean * mean
    inv = 1.0 / jnp.sqrt(var + 1e-5)
    hbn = (h_ref[...] - mean) * inv
    cnt = jnp.sum(c_ref[...], axis=0)[:, 0:1]
    aggr = jnp.sum(p_ref[...], axis=0) / jnp.maximum(cnt, 1.0)
    aggbn = jnp.where(cnt > 0.0, (aggr - mean) * inv, 0.0)
    out = _mm_t(aggbn, wl_ref[...]) + bl_ref[...] + _mm_t(hbn, wr_ref[...])
    nrm = jnp.sqrt(jnp.sum(out * out, axis=1, keepdims=True))
    o_ref[...] = out / jnp.maximum(nrm, 1e-12)


def _tc_layer2(h, p, c, stats, wl, bl, wr):
    return pl.pallas_call(
        _tc2_body,
        grid=(NBLK,),
        in_specs=[
            pl.BlockSpec((BLK, NF), lambda i: (i, 0)),
            pl.BlockSpec((NC, BLK, NF), lambda i: (0, i, 0)),
            pl.BlockSpec((NC, BLK, NF), lambda i: (0, i, 0)),
            pl.BlockSpec((8, NF), lambda i: (0, 0)),
            pl.BlockSpec((NF, NF), lambda i: (0, 0)),
            pl.BlockSpec((1, NF), lambda i: (0, 0)),
            pl.BlockSpec((NF, NF), lambda i: (0, 0)),
        ],
        out_specs=pl.BlockSpec((BLK, NF), lambda i: (i, 0)),
        out_shape=jax.ShapeDtypeStruct((N, NF), jnp.float32),
    )(h, p, c, stats, wl, bl, wr)


def kernel(x, edge_index, W1l, b1l, W1r, W2l, b2l, W2r):
    src3 = edge_index[0].astype(jnp.int32).reshape(NW, NG, GP, CH)
    dst3 = edge_index[1].astype(jnp.int32).reshape(NW, NG, GP, CH)
    b1 = jnp.reshape(b1l, (1, -1))
    b2 = jnp.reshape(b2l, (1, -1))
    z128 = jnp.zeros((RB, NF), jnp.float32)
    ones128 = jnp.ones((CH, NF), jnp.float32)
    p1, c1 = _sc_segsum(x, src3, dst3, z128, ones128)
    h, stats = _tc_layer1(x, p1, c1, W1l, b1, W1r)
    p2 = _sc_segsum(h, src3, dst3, z128)
    return _tc_layer2(h, p2, c1, stats, W2l, b2, W2r)


# speedup vs baseline: 11.0158x; 1.0083x over previous
import functools

import jax
import jax.numpy as jnp
from jax import lax
from jax.experimental import pallas as pl
from jax.experimental.pallas import tpu as pltpu
from jax.experimental.pallas import tpu_sc as plsc

N = 10000
NF = 128
E = 320000
NC = 2
NS = 16
NW = NC * NS
EW = E // NW
CH = 80
NCHUNK = EW // CH
GP = 25
NG = NCHUNK // GP
RB = 632
RB_LAST = N - (NS - 1) * RB
NBLK = 2
BLK = N // NBLK


def _per_tile_rows(s, fn):
    @pl.when(s < NS - 1)
    def _():
        fn(pl.multiple_of(s * RB, 8), RB)

    @pl.when(s == NS - 1)
    def _():
        fn((NS - 1) * RB, RB_LAST)


def _sc_segsum_body(with_cnt, *refs):
    if with_cnt:
        (x_hbm, src_hbm, dst_hbm, z_hbm, ones_hbm, pagg, pcnt,
         sidx, didx, rows_a, rows_b, rows_c, acc, ga, gb, gc, sa, sb, sc) = refs
    else:
        (x_hbm, src_hbm, dst_hbm, z_hbm, pagg,
         sidx, didx, rows_a, rows_b, rows_c, acc, ga, gb, gc, sa, sb, sc) = refs
    c = lax.axis_index("c")
    s = lax.axis_index("s")
    wid = s * NC + c

    _per_tile_rows(s, lambda rb, nr: pltpu.sync_copy(
        z_hbm.at[pl.ds(0, nr)], acc.at[pl.ds(rb, nr)]))
    plsc.subcore_barrier()

    if with_cnt:
        pltpu.sync_copy(ones_hbm, rows_c)

        def cgroup(g, _):
            pltpu.sync_copy(dst_hbm.at[wid, g], didx)

            def fire(j, _):
                pltpu.async_copy(rows_c, acc.at[didx.at[j]], sa, add=True)
                return 0

            def steady(j, _):
                pltpu.make_async_copy(rows_c, acc.at[didx.at[0]], sa).wait()
                pltpu.async_copy(rows_c, acc.at[didx.at[j]], sa, add=True)
                return 0

            def cdrain(j, _):
                pltpu.make_async_copy(rows_c, acc.at[didx.at[0]], sa).wait()
                return 0

            lax.fori_loop(0, 8, fire, 0)
            lax.fori_loop(8, GP, steady, 0)
            lax.fori_loop(0, 8, cdrain, 0)
            return 0

        lax.fori_loop(0, NG, cgroup, 0)
        plsc.subcore_barrier()
        _per_tile_rows(s, lambda rb, nr: pltpu.sync_copy(
            acc.at[pl.ds(rb, nr)], pcnt.at[c, pl.ds(rb, nr)]))
        plsc.subcore_barrier()

    def gwait(rows, sem):
        pltpu.make_async_copy(x_hbm.at[pl.ds(0, CH)], rows, sem).wait()

    def swait(rows, sem):
        pltpu.make_async_copy(rows, acc.at[didx.at[0]], sem).wait()

    def gather(j, rows, sem):
        pltpu.async_copy(x_hbm.at[sidx.at[j]], rows, sem)

    def scat(j, rows, sem):
        pltpu.async_copy(rows, acc.at[didx.at[j]], sem, add=True)

    def group(g, _):
        pltpu.sync_copy(src_hbm.at[wid, g], sidx)
        pltpu.sync_copy(dst_hbm.at[wid, g], didx)
        gather(0, rows_a, ga)
        gather(1, rows_b, gb)
        gwait(rows_a, ga)
        scat(0, rows_a, sa)
        gather(2, rows_c, gc)

        def tri(k, _):
            j = 3 * k + 1
            gwait(rows_b, gb)
            scat(j, rows_b, sb)
            swait(rows_a, sa)
            gather(j + 2, rows_a, ga)
            gwait(rows_c, gc)
            scat(j + 1, rows_c, sc)
            swait(rows_b, sb)
            gather(j + 3, rows_b, gb)
            gwait(rows_a, ga)
            scat(j + 2, rows_a, sa)
            swait(rows_c, sc)
            gather(j + 4, rows_c, gc)
            return 0

        lax.fori_loop(0, (GP - 4) // 3, tri, 0)
        gwait(rows_b, gb)
        scat(GP - 3, rows_b, sb)
        swait(rows_a, sa)
        gather(GP - 1, rows_a, ga)
        gwait(rows_c, gc)
        scat(GP - 2, rows_c, sc)
        swait(rows_b, sb)
        gwait(rows_a, ga)
        scat(GP - 1, rows_a, sa)
        swait(rows_c, sc)
        swait(rows_a, sa)
        return 0

    lax.fori_loop(0, NG, group, 0)
    plsc.subcore_barrier()

    _per_tile_rows(s, lambda rb, nr: pltpu.sync_copy(
        acc.at[pl.ds(rb, nr)], pagg.at[c, pl.ds(rb, nr)]))


def _sc_segsum(x, src4, dst4, z128, ones128=None):
    with_cnt = ones128 is not None
    if with_cnt:
        out_type = [jax.ShapeDtypeStruct((NC, N, NF), jnp.float32),
                    jax.ShapeDtypeStruct((NC, N, NF), jnp.float32)]
    else:
        out_type = jax.ShapeDtypeStruct((NC, N, NF), jnp.float32)
    fn = pl.kernel(
        functools.partial(_sc_segsum_body, with_cnt),
        out_type=out_type,
        mesh=plsc.VectorSubcoreMesh(core_axis_name="c", subcore_axis_name="s"),
        scratch_types=[
            pltpu.VMEM((GP, CH), jnp.int32),
            pltpu.VMEM((GP, CH), jnp.int32),
            pltpu.VMEM((CH, NF), jnp.float32),
            pltpu.VMEM((CH, NF), jnp.float32),
            pltpu.VMEM((CH, NF), jnp.float32),
            pltpu.VMEM_SHARED((N, NF), jnp.float32),
            pltpu.SemaphoreType.DMA,
            pltpu.SemaphoreType.DMA,
            pltpu.SemaphoreType.DMA,
            pltpu.SemaphoreType.DMA,
            pltpu.SemaphoreType.DMA,
            pltpu.SemaphoreType.DMA,
        ],
    )
    if with_cnt:
        return fn(x, src4, dst4, z128, ones128)
    return fn(x, src4, dst4, z128)


def _mm_t(a, w):
    return jax.lax.dot_general(a, w, (((1,), (1,)), ((), ())),
                               preferred_element_type=jnp.float32)


def _tc1_body(x_ref, p_ref, c_ref, wl_ref, bl_ref, wr_ref,
              h_ref, st_ref, sacc):
    i = pl.program_id(0)
    cnt = jnp.sum(c_ref[...], axis=0)[:, 0:1]
    agg = (jnp.sum(p_ref[...], axis=0) - cnt) / jnp.maximum(cnt, 1.0)
    out = _mm_t(agg, wl_ref[...]) + bl_ref[...] + _mm_t(x_ref[...], wr_ref[...])
    nrm = jnp.sqrt(jnp.sum(out * out, axis=1, keepdims=True))
    h = jnp.maximum(out / jnp.maximum(nrm, 1e-12), 0.0)
    h_ref[...] = h
    s0 = jnp.sum(h, axis=0, keepdims=True)
    s1 = jnp.sum(h * h, axis=0, keepdims=True)
    upd = jnp.concatenate([s0, s1, jnp.zeros((6, NF), jnp.float32)], axis=0)

    @pl.when(i == 0)
    def _():
        sacc[...] = upd

    @pl.when(i > 0)
    def _():
        sacc[...] = sacc[...] + upd

    @pl.when(i == NBLK - 1)
    def _():
        st_ref[...] = sacc[...]


def _tc_layer1(x, p, c, wl, bl, wr):
    return pl.pallas_call(
        _tc1_body,
        grid=(NBLK,),
        in_specs=[
            pl.BlockSpec((BLK, NF), lambda i: (i, 0)),
            pl.BlockSpec((NC, BLK, NF), lambda i: (0, i, 0)),
            pl.BlockSpec((NC, BLK, NF), lambda i: (0, i, 0)),
            pl.BlockSpec((NF, NF), lambda i: (0, 0)),
            pl.BlockSpec((1, NF), lambda i: (0, 0)),
            pl.BlockSpec((NF, NF), lambda i: (0, 0)),
        ],
        out_specs=[
            pl.BlockSpec((BLK, NF), lambda i: (i, 0)),
            pl.BlockSpec((8, NF), lambda i: (0, 0)),
        ],
        out_shape=[
            jax.ShapeDtypeStruct((N, NF), jnp.float32),
            jax.ShapeDtypeStruct((8, NF), jnp.float32),
        ],
        scratch_shapes=[pltpu.VMEM((8, NF), jnp.float32)],
    )(x, p, c, wl, bl, wr)


def _tc2_body(h_ref, p_ref, c_ref, st_ref, wl_ref, bl_ref, wr_ref, o_ref):
    stats = st_ref[...]
    mean = stats[0:1, :] * (1.0 / N)
    var = stats[1:2, :] * (1.0 / N) - mean * mean
    inv = 1.0 / jnp.sqrt(var + 1e-5)
    hbn = (h_ref[...] - mean) * inv
    cnt = jnp.sum(c_ref[...], axis=0)[:, 0:1]
    aggr = jnp.sum(p_ref[...], axis=0) / jnp.maximum(cnt, 1.0)
    aggbn = jnp.where(cnt > 0.0, (aggr - mean) * inv, 0.0)
    out = _mm_t(aggbn, wl_ref[...]) + bl_ref[...] + _mm_t(hbn, wr_ref[...])
    nrm = jnp.sqrt(jnp.sum(out * out, axis=1, keepdims=True))
    o_ref[...] = out / jnp.maximum(nrm, 1e-12)


def _tc_layer2(h, p, c, stats, wl, bl, wr):
    return pl.pallas_call(
        _tc2_body,
        grid=(NBLK,),
        in_specs=[
            pl.BlockSpec((BLK, NF), lambda i: (i, 0)),
            pl.BlockSpec((NC, BLK, NF), lambda i: (0, i, 0)),
            pl.BlockSpec((NC, BLK, NF), lambda i: (0, i, 0)),
            pl.BlockSpec((8, NF), lambda i: (0, 0)),
            pl.BlockSpec((NF, NF), lambda i: (0, 0)),
            pl.BlockSpec((1, NF), lambda i: (0, 0)),
            pl.BlockSpec((NF, NF), lambda i: (0, 0)),
        ],
        out_specs=pl.BlockSpec((BLK, NF), lambda i: (i, 0)),
        out_shape=jax.ShapeDtypeStruct((N, NF), jnp.float32),
    )(h, p, c, stats, wl, bl, wr)


def kernel(x, edge_index, W1l, b1l, W1r, W2l, b2l, W2r):
    src3 = edge_index[0].astype(jnp.int32).reshape(NW, NG, GP, CH)
    dst3 = edge_index[1].astype(jnp.int32).reshape(NW, NG, GP, CH)
    b1 = jnp.reshape(b1l, (1, -1))
    b2 = jnp.reshape(b2l, (1, -1))
    z128 = jnp.zeros((RB, NF), jnp.float32)
    ones128 = jnp.ones((CH, NF), jnp.float32)
    p1, c1 = _sc_segsum(x, src3, dst3, z128, ones128)
    h, stats = _tc_layer1(x, p1, c1, W1l, b1, W1r)
    p2 = _sc_segsum(h, src3, dst3, z128)
    return _tc_layer2(h, p2, c1, stats, W2l, b2, W2r)
